# baseline (speedup 1.0000x reference)
import math

import jax
import jax.numpy as jnp
from jax.experimental import pallas as pl

BQ = 256
HP = 2


def _attn_block_kernel(q_ref, k_ref, v_ref, m_ref, o_ref):
    dh = q_ref.shape[-1] // HP
    c = math.log2(math.e) / math.sqrt(dh)
    qp = (q_ref[...] * c).astype(jnp.bfloat16)
    kp = k_ref[...].astype(jnp.bfloat16)
    vp = v_ref[...].astype(jnp.bfloat16)
    outs = []
    for j in range(HP):
        qj = qp[:, j * dh:(j + 1) * dh]
        kj = kp[:, j * dh:(j + 1) * dh]
        vj = vp[:, j * dh:(j + 1) * dh]
        mj = m_ref[j]
        s2 = jax.lax.dot_general(
            qj, kj, (((1,), (1,)), ((), ())), preferred_element_type=jnp.float32
        )
        mx = jnp.max(s2, axis=-1, keepdims=True)
        p = jnp.exp2(s2 - mx) * mj
        d = jnp.sum(p, axis=-1, keepdims=True)
        o = jax.lax.dot_general(
            p.astype(jnp.bfloat16), vj, (((1,), (0,)), ((), ())),
            preferred_element_type=jnp.float32,
        )
        outs.append(o / d)
    o_ref[...] = jnp.concatenate(outs, axis=-1)


def kernel(query, key, value, mask):
    b, s, h, dh = query.shape
    hd = h * dh
    nq = s // BQ
    nh = h // HP

    qb = query.reshape(s, hd)
    kb = key.reshape(s, hd)
    vb = value.reshape(s, hd)

    out = pl.pallas_call(
        _attn_block_kernel,
        grid=(nh, nq),
        in_specs=[
            pl.BlockSpec((BQ, HP * dh), lambda hh, i: (i, hh)),
            pl.BlockSpec((s, HP * dh), lambda hh, i: (0, hh)),
            pl.BlockSpec((s, HP * dh), lambda hh, i: (0, hh)),
            pl.BlockSpec((HP, BQ, s), lambda hh, i: (hh, i, 0)),
        ],
        out_specs=pl.BlockSpec((BQ, HP * dh), lambda hh, i: (i, hh)),
        out_shape=jax.ShapeDtypeStruct((s, hd), jnp.float32),
    )(qb, kb, vb, mask)

    return out.reshape(s, b, hd)
